# baseline (speedup 1.0000x reference)
import functools

import jax
import jax.numpy as jnp
from jax import lax
from jax.experimental import pallas as pl
from jax.experimental.pallas import tpu as pltpu
from jax.experimental.pallas import tpu_sc as plsc

ALPHA_THRESHOLD = 1.0 / 255.0
MAX_ALPHA = 0.99
EPS = 1e-8
PIX_OFF = 0.5
H = 128
W = 128
G = 2048
KC = 8
NSTRIP = 16
SH = H // NSTRIP
NSEG = 2
NWORK = NSTRIP * NSEG
SEGG = G // NSEG
CAP = SEGG
NC = 2
LANES = 16


def _sc_bin_body(my_h, op_h, a_h, b_h, c_h, order_h, params_h,
                 gp_h, counts_h,
                 my_v, op_v, a_v, b_v, c_v, ord_v, idx_v, rows_v, cnt_v, sem):
    wid = lax.axis_index("s") * NC + lax.axis_index("c")
    strip = wid // NSEG
    seg = wid % NSEG

    pltpu.sync_copy(my_h, my_v)
    pltpu.sync_copy(op_h, op_v)
    pltpu.sync_copy(a_h, a_v)
    pltpu.sync_copy(b_h, b_v)
    pltpu.sync_copy(c_h, c_v)
    pltpu.sync_copy(order_h.at[pl.ds(seg * SEGG, SEGG)], ord_v)

    ylo_c = strip.astype(jnp.float32) * float(SH) + PIX_OFF
    yhi_c = ylo_c + float(SH - 1)

    def zero_body(i, _):
        idx_v[i // 8, pl.ds((i % 8) * LANES, LANES)] = jnp.zeros((LANES,), jnp.int32)
        return 0

    lax.fori_loop(0, CAP // LANES, zero_body, 0)

    def scan_body(i, cnt):
        ids = ord_v[pl.ds(i * LANES, LANES)]
        myv = plsc.load_gather(my_v, [ids])
        opv = plsc.load_gather(op_v, [ids])
        av = plsc.load_gather(a_v, [ids])
        bv = plsc.load_gather(b_v, [ids])
        cv = plsc.load_gather(c_v, [ids])
        det = av * cv - bv * bv
        wq = det / jnp.where(av > 0.0, av, 1.0)
        dy = jnp.clip(myv, ylo_c, yhi_c) - myv
        amax = opv * jnp.exp(-0.5 * (dy * dy * wq))
        valid = (opv > ALPHA_THRESHOLD) & (det > EPS) & (av > 0.0) & (cv > 0.0)
        m = valid & (amax >= ALPHA_THRESHOLD * 0.999)
        pos = cnt + plsc.cumsum(m.astype(jnp.int32)) - 1
        plsc.store_scatter(idx_v, [lax.div(pos, 128), lax.rem(pos, 128)], ids, mask=m)
        return cnt + plsc.all_reduce_population_count(m)

    cnt = lax.fori_loop(0, SEGG // LANES, scan_body, jnp.zeros((LANES,), jnp.int32))
    cnt_v[...] = cnt
    pltpu.sync_copy(cnt_v, counts_h.at[wid])

    copies = [pltpu.async_copy(params_h.at[idx_v.at[j]], rows_v.at[j], sem)
              for j in range(CAP // 128)]
    for cp in copies:
        cp.wait()
    pltpu.sync_copy(rows_v, gp_h.at[wid])


_sc_bin = functools.partial(
    pl.kernel,
    out_type=(
        jax.ShapeDtypeStruct((NWORK, CAP // 128, 128, 16), jnp.float32),
        jax.ShapeDtypeStruct((NWORK, LANES), jnp.int32),
    ),
    mesh=plsc.VectorSubcoreMesh(core_axis_name="c", subcore_axis_name="s"),
    compiler_params=pltpu.CompilerParams(
        needs_layout_passes=False, use_tc_tiling_on_sc=False),
    scratch_types=[
        pltpu.VMEM((G,), jnp.float32),
        pltpu.VMEM((G,), jnp.float32),
        pltpu.VMEM((G,), jnp.float32),
        pltpu.VMEM((G,), jnp.float32),
        pltpu.VMEM((G,), jnp.float32),
        pltpu.VMEM((SEGG,), jnp.int32),
        pltpu.VMEM((CAP // 128, 128), jnp.int32),
        pltpu.VMEM((CAP // 128, 128, 16), jnp.float32),
        pltpu.VMEM((LANES,), jnp.int32),
        pltpu.SemaphoreType.DMA,
    ],
)(_sc_bin_body)


def _tc_comp_body(counts_ref, bg_ref, gp_ref, out_ref, accr, accg, accb, trans_ref):
    i = pl.program_id(0)
    strip = i // NSEG
    seg = lax.rem(i, NSEG)

    @pl.when(seg == 0)
    def _init():
        accr[:, :] = jnp.zeros((SH, W), jnp.float32)
        accg[:, :] = jnp.zeros((SH, W), jnp.float32)
        accb[:, :] = jnp.zeros((SH, W), jnp.float32)
        trans_ref[:, :] = jnp.ones((SH, W), jnp.float32)

    count = counts_ref[i, 0]
    xs = jax.lax.broadcasted_iota(jnp.int32, (1, 1, W), 2).astype(jnp.float32) + PIX_OFF
    ys = (jax.lax.broadcasted_iota(jnp.int32, (1, SH, 1), 1) + strip * SH
          ).astype(jnp.float32) + PIX_OFF

    def chunk(j, _):
        p = gp_ref[0, pl.ds(j * KC, KC), :]
        mx = p[:, 0:1][:, :, None]
        my = p[:, 1:2][:, :, None]
        a = p[:, 2:3][:, :, None]
        b = p[:, 3:4][:, :, None]
        c = p[:, 4:5][:, :, None]
        op = p[:, 5:6][:, :, None]

        det = a * c - b * b
        tau = -2.0 * jnp.log(jnp.maximum(ALPHA_THRESHOLD / jnp.maximum(op, EPS), EPS))
        vmask = (op > ALPHA_THRESHOLD) & (det > EPS) & (a > 0.0) & (c > 0.0) & (tau > 0.0)
        rowmask = (j * KC + jax.lax.broadcasted_iota(jnp.int32, (KC, 1, 1), 0)) < count

        dx = xs - mx
        dy = ys - my
        q = a * (dx * dx) + 2.0 * b * (dx * dy) + c * (dy * dy)
        alpha = jnp.where((q <= tau) & vmask & rowmask, op * jnp.exp(-0.5 * q), 0.0)
        alpha = jnp.minimum(alpha, MAX_ALPHA)

        t = trans_ref[:, :]
        ws = []
        for g in range(KC):
            ag = alpha[g]
            ws.append(t * ag)
            t = t * (1.0 - ag)
        wstack = jnp.stack(ws, axis=0)

        cr = p[:, 6:7][:, :, None]
        cg = p[:, 7:8][:, :, None]
        cb = p[:, 8:9][:, :, None]
        accr[:, :] += jnp.sum(wstack * cr, axis=0)
        accg[:, :] += jnp.sum(wstack * cg, axis=0)
        accb[:, :] += jnp.sum(wstack * cb, axis=0)
        trans_ref[:, :] = t
        return 0

    lax.fori_loop(0, lax.div(count + (KC - 1), KC), chunk, 0)

    @pl.when(seg == NSEG - 1)
    def _fin():
        tt = trans_ref[:, :]
        out_ref[0, :, :] = accr[:, :] + tt * bg_ref[0]
        out_ref[1, :, :] = accg[:, :] + tt * bg_ref[1]
        out_ref[2, :, :] = accb[:, :] + tt * bg_ref[2]


def kernel(means2d, conics, colors, opacities, depths, background, image_height, image_width):
    order = jnp.argsort(lax.stop_gradient(depths)).astype(jnp.int32)
    params = jnp.zeros((G, 16), jnp.float32)
    params = params.at[:, 0:2].set(means2d)
    params = params.at[:, 2:5].set(conics)
    params = params.at[:, 5].set(opacities)
    params = params.at[:, 6:9].set(colors)

    gp, counts = _sc_bin(
        means2d[:, 1], opacities,
        conics[:, 0], conics[:, 1], conics[:, 2],
        order, params,
    )
    gp = gp.reshape(NWORK, CAP, 16)

    out = pl.pallas_call(
        _tc_comp_body,
        grid=(NWORK,),
        in_specs=[
            pl.BlockSpec(memory_space=pltpu.SMEM),
            pl.BlockSpec(memory_space=pltpu.SMEM),
            pl.BlockSpec((1, CAP, 16), lambda i: (i, 0, 0)),
        ],
        out_specs=pl.BlockSpec((3, SH, W), lambda i: (0, i // NSEG, 0)),
        out_shape=jax.ShapeDtypeStruct((3, H, W), jnp.float32),
        scratch_shapes=[
            pltpu.VMEM((SH, W), jnp.float32),
            pltpu.VMEM((SH, W), jnp.float32),
            pltpu.VMEM((SH, W), jnp.float32),
            pltpu.VMEM((SH, W), jnp.float32),
        ],
    )(counts, background.astype(jnp.float32), gp)
    return jnp.transpose(out, (1, 2, 0)).astype(means2d.dtype)


# speedup vs baseline: 1.4442x; 1.4442x over previous
import functools

import jax
import jax.numpy as jnp
from jax import lax
from jax.experimental import pallas as pl
from jax.experimental.pallas import tpu as pltpu
from jax.experimental.pallas import tpu_sc as plsc

ALPHA_THRESHOLD = 1.0 / 255.0
MAX_ALPHA = 0.99
EPS = 1e-8
PIX_OFF = 0.5
H = 128
W = 128
G = 2048
KC = 8
NSTRIP = 16
SH = H // NSTRIP
NSEG = 2
NWORK = NSTRIP * NSEG
SEGG = G // NSEG
CAP = SEGG
NC = 2
LANES = 16
GB = CAP // 128


def _tc_prep_body(pt_ref, prep_ref):
    my = pt_ref[1:2, :]
    a = pt_ref[2:3, :]
    b = pt_ref[3:4, :]
    c = pt_ref[4:5, :]
    op = pt_ref[5:6, :]
    det = a * c - b * b
    valid = (op > ALPHA_THRESHOLD) & (det > EPS) & (a > 0.0) & (c > 0.0)
    tau = -2.0 * jnp.log(jnp.maximum(ALPHA_THRESHOLD / jnp.maximum(op, EPS), EPS))
    valid = valid & (tau > 0.0)
    wq = jnp.where(valid, det / jnp.maximum(a, EPS), 0.0)
    tau_b = jnp.where(valid, tau * 1.001 + 1e-5, -1.0)
    prep_ref[0:1, :] = my
    prep_ref[1:2, :] = wq
    prep_ref[2:3, :] = tau_b


def _sc_bin_body(prep_h, params_h, gp_h, counts_h,
                 my_v, wq_v, tau_v, idx_v, rows_v, cnt_v, sem):
    wid = lax.axis_index("s") * NC + lax.axis_index("c")
    strip = wid // NSEG
    seg = wid % NSEG

    base = seg * SEGG
    pltpu.sync_copy(prep_h.at[0, pl.ds(base, SEGG)], my_v)
    pltpu.sync_copy(prep_h.at[1, pl.ds(base, SEGG)], wq_v)
    pltpu.sync_copy(prep_h.at[2, pl.ds(base, SEGG)], tau_v)

    ylo_c = strip.astype(jnp.float32) * float(SH) + PIX_OFF
    yhi_c = ylo_c + float(SH - 1)

    def zero_body(i, _):
        idx_v[i // 8, pl.ds((i % 8) * LANES, LANES)] = jnp.zeros((LANES,), jnp.int32)
        return 0

    lax.fori_loop(0, CAP // LANES, zero_body, 0)

    lane = lax.iota(jnp.int32, LANES) + base

    def scan_body(i, cnt):
        sl = pl.ds(i * LANES, LANES)
        myv = my_v[sl]
        wqv = wq_v[sl]
        tauv = tau_v[sl]
        dy = jnp.clip(myv, ylo_c, yhi_c) - myv
        m = (dy * dy) * wqv <= tauv
        pos = cnt + plsc.cumsum(m.astype(jnp.int32)) - 1
        ids = lane + i * LANES
        plsc.store_scatter(idx_v, [lax.div(pos, 128), lax.rem(pos, 128)], ids, mask=m)
        return cnt + plsc.all_reduce_population_count(m)

    cnt = lax.fori_loop(0, SEGG // LANES, scan_body, jnp.zeros((LANES,), jnp.int32))
    cnt_v[...] = cnt
    count = jnp.max(cnt)
    pltpu.sync_copy(cnt_v, counts_h.at[wid])

    for j in range(GB):
        @pl.when(count > j * 128)
        def _gather(j=j):
            pltpu.async_copy(params_h.at[idx_v.at[j]], rows_v.at[j], sem).wait()
            pltpu.sync_copy(rows_v.at[j], gp_h.at[wid, j])


_sc_bin = functools.partial(
    pl.kernel,
    out_type=(
        jax.ShapeDtypeStruct((NWORK, GB, 128, 16), jnp.float32),
        jax.ShapeDtypeStruct((NWORK, LANES), jnp.int32),
    ),
    mesh=plsc.VectorSubcoreMesh(core_axis_name="c", subcore_axis_name="s"),
    compiler_params=pltpu.CompilerParams(
        needs_layout_passes=False, use_tc_tiling_on_sc=False),
    scratch_types=[
        pltpu.VMEM((SEGG,), jnp.float32),
        pltpu.VMEM((SEGG,), jnp.float32),
        pltpu.VMEM((SEGG,), jnp.float32),
        pltpu.VMEM((GB, 128), jnp.int32),
        pltpu.VMEM((GB, 128, 16), jnp.float32),
        pltpu.VMEM((LANES,), jnp.int32),
        pltpu.SemaphoreType.DMA,
    ],
)(_sc_bin_body)


def _tc_comp_body(counts_ref, bg_ref, gp_ref, out_ref, accr, accg, accb, trans_ref):
    i = pl.program_id(0)
    strip = i // NSEG
    seg = lax.rem(i, NSEG)

    @pl.when(seg == 0)
    def _init():
        accr[:, :] = jnp.zeros((SH, W), jnp.float32)
        accg[:, :] = jnp.zeros((SH, W), jnp.float32)
        accb[:, :] = jnp.zeros((SH, W), jnp.float32)
        trans_ref[:, :] = jnp.ones((SH, W), jnp.float32)

    count = counts_ref[i, 0]
    xs = jax.lax.broadcasted_iota(jnp.int32, (1, 1, W), 2).astype(jnp.float32) + PIX_OFF
    ys = (jax.lax.broadcasted_iota(jnp.int32, (1, SH, 1), 1) + strip * SH
          ).astype(jnp.float32) + PIX_OFF

    def chunk(j, _):
        p = gp_ref[0, pl.ds(j * KC, KC), :]
        mx = p[:, 0:1][:, :, None]
        my = p[:, 1:2][:, :, None]
        a = p[:, 2:3][:, :, None]
        b = p[:, 3:4][:, :, None]
        c = p[:, 4:5][:, :, None]
        op = p[:, 5:6][:, :, None]

        det = a * c - b * b
        tau = -2.0 * jnp.log(jnp.maximum(ALPHA_THRESHOLD / jnp.maximum(op, EPS), EPS))
        vmask = (op > ALPHA_THRESHOLD) & (det > EPS) & (a > 0.0) & (c > 0.0) & (tau > 0.0)
        rowmask = (j * KC + jax.lax.broadcasted_iota(jnp.int32, (KC, 1, 1), 0)) < count

        dx = xs - mx
        dy = ys - my
        q = a * (dx * dx) + 2.0 * b * (dx * dy) + c * (dy * dy)
        alpha = jnp.where((q <= tau) & vmask & rowmask, op * jnp.exp(-0.5 * q), 0.0)
        alpha = jnp.minimum(alpha, MAX_ALPHA)

        t = trans_ref[:, :]
        ws = []
        for g in range(KC):
            ag = alpha[g]
            ws.append(t * ag)
            t = t * (1.0 - ag)
        wstack = jnp.stack(ws, axis=0)

        cr = p[:, 6:7][:, :, None]
        cg = p[:, 7:8][:, :, None]
        cb = p[:, 8:9][:, :, None]
        accr[:, :] += jnp.sum(wstack * cr, axis=0)
        accg[:, :] += jnp.sum(wstack * cg, axis=0)
        accb[:, :] += jnp.sum(wstack * cb, axis=0)
        trans_ref[:, :] = t
        return 0

    lax.fori_loop(0, lax.div(count + (KC - 1), KC), chunk, 0)

    @pl.when(seg == NSEG - 1)
    def _fin():
        tt = trans_ref[:, :]
        out_ref[0, :, :] = accr[:, :] + tt * bg_ref[0]
        out_ref[1, :, :] = accg[:, :] + tt * bg_ref[1]
        out_ref[2, :, :] = accb[:, :] + tt * bg_ref[2]


def kernel(means2d, conics, colors, opacities, depths, background, image_height, image_width):
    order = jnp.argsort(lax.stop_gradient(depths))
    params = jnp.zeros((G, 16), jnp.float32)
    params = params.at[:, 0:2].set(means2d)
    params = params.at[:, 2:5].set(conics)
    params = params.at[:, 5].set(opacities)
    params = params.at[:, 6:9].set(colors)
    params = jnp.take(params, order, axis=0)

    prep = pl.pallas_call(
        _tc_prep_body,
        in_specs=[pl.BlockSpec((16, G), lambda: (0, 0))],
        out_specs=pl.BlockSpec((8, G), lambda: (0, 0)),
        out_shape=jax.ShapeDtypeStruct((8, G), jnp.float32),
    )(params.T)

    gp, counts = _sc_bin(prep, params)
    gp = gp.reshape(NWORK, CAP, 16)

    out = pl.pallas_call(
        _tc_comp_body,
        grid=(NWORK,),
        in_specs=[
            pl.BlockSpec(memory_space=pltpu.SMEM),
            pl.BlockSpec(memory_space=pltpu.SMEM),
            pl.BlockSpec((1, CAP, 16), lambda i: (i, 0, 0)),
        ],
        out_specs=pl.BlockSpec((3, SH, W), lambda i: (0, i // NSEG, 0)),
        out_shape=jax.ShapeDtypeStruct((3, H, W), jnp.float32),
        scratch_shapes=[
            pltpu.VMEM((SH, W), jnp.float32),
            pltpu.VMEM((SH, W), jnp.float32),
            pltpu.VMEM((SH, W), jnp.float32),
            pltpu.VMEM((SH, W), jnp.float32),
        ],
    )(counts, background.astype(jnp.float32), gp)
    return jnp.transpose(out, (1, 2, 0)).astype(means2d.dtype)


# speedup vs baseline: 1.9729x; 1.3661x over previous
import functools

import jax
import jax.numpy as jnp
from jax import lax
from jax.experimental import pallas as pl
from jax.experimental.pallas import tpu as pltpu
from jax.experimental.pallas import tpu_sc as plsc

ALPHA_THRESHOLD = 1.0 / 255.0
MAX_ALPHA = 0.99
EPS = 1e-8
PIX_OFF = 0.5
H = 128
W = 128
G = 2048
KC = 16
NSTRIP = 16
SH = H // NSTRIP
NSEG = 2
NWORK = NSTRIP * NSEG
SEGG = G // NSEG
CAP = SEGG
NC = 2
LANES = 16
GB = CAP // 128


def _tc_prep_body(pt_ref, prep_ref):
    a = pt_ref[2:3, :]
    b = pt_ref[3:4, :]
    c = pt_ref[4:5, :]
    op = pt_ref[5:6, :]
    det = a * c - b * b
    valid = (op > ALPHA_THRESHOLD) & (det > EPS) & (a > 0.0) & (c > 0.0)
    tau = -2.0 * jnp.log(jnp.maximum(ALPHA_THRESHOLD / jnp.maximum(op, EPS), EPS))
    valid = valid & (tau > 0.0)
    wq = jnp.where(valid, det / jnp.maximum(a, EPS), 0.0)
    tau_b = jnp.where(valid, tau * 1.001 + 1e-5, -1.0)
    prep_ref[0:9, :] = pt_ref[0:9, :]
    prep_ref[9:10, :] = jnp.where(valid, tau, -1.0)
    prep_ref[10:16, :] = jnp.zeros((6, G), jnp.float32)
    prep_ref[16:17, :] = wq
    prep_ref[17:18, :] = tau_b
    prep_ref[18:24, :] = jnp.zeros((6, G), jnp.float32)


def _sc_bin_body(prep_h, params_h, gp_h, counts_h,
                 my_v, wq_v, tau_v, idx_v, rows_v, cnt_v, sem):
    wid = lax.axis_index("s") * NC + lax.axis_index("c")
    strip = wid // NSEG
    seg = wid % NSEG

    base = seg * SEGG
    pltpu.sync_copy(prep_h.at[1, pl.ds(base, SEGG)], my_v)
    pltpu.sync_copy(prep_h.at[16, pl.ds(base, SEGG)], wq_v)
    pltpu.sync_copy(prep_h.at[17, pl.ds(base, SEGG)], tau_v)

    ylo_c = strip.astype(jnp.float32) * float(SH) + PIX_OFF
    yhi_c = ylo_c + float(SH - 1)

    def zero_body(i, _):
        idx_v[i // 8, pl.ds((i % 8) * LANES, LANES)] = jnp.zeros((LANES,), jnp.int32)
        return 0

    lax.fori_loop(0, CAP // LANES, zero_body, 0)

    lane = lax.iota(jnp.int32, LANES) + base

    def scan_body(i, cnt):
        sl = pl.ds(i * LANES, LANES)
        myv = my_v[sl]
        wqv = wq_v[sl]
        tauv = tau_v[sl]
        dy = jnp.clip(myv, ylo_c, yhi_c) - myv
        m = (dy * dy) * wqv <= tauv
        pos = cnt + plsc.cumsum(m.astype(jnp.int32)) - 1
        ids = lane + i * LANES
        plsc.store_scatter(idx_v, [lax.div(pos, 128), lax.rem(pos, 128)], ids, mask=m)
        return cnt + plsc.all_reduce_population_count(m)

    cnt = lax.fori_loop(0, SEGG // LANES, scan_body, jnp.zeros((LANES,), jnp.int32))
    cnt_v[...] = cnt
    count = jnp.max(cnt)
    pltpu.sync_copy(cnt_v, counts_h.at[wid])

    for j in range(GB):
        @pl.when(count > j * 128)
        def _gather(j=j):
            pltpu.async_copy(params_h.at[idx_v.at[j]], rows_v.at[j], sem).wait()
            pltpu.sync_copy(rows_v.at[j], gp_h.at[wid, j])


_sc_bin = functools.partial(
    pl.kernel,
    out_type=(
        jax.ShapeDtypeStruct((NWORK, GB, 128, 16), jnp.float32),
        jax.ShapeDtypeStruct((NWORK, LANES), jnp.int32),
    ),
    mesh=plsc.VectorSubcoreMesh(core_axis_name="c", subcore_axis_name="s"),
    compiler_params=pltpu.CompilerParams(
        needs_layout_passes=False, use_tc_tiling_on_sc=False),
    scratch_types=[
        pltpu.VMEM((SEGG,), jnp.float32),
        pltpu.VMEM((SEGG,), jnp.float32),
        pltpu.VMEM((SEGG,), jnp.float32),
        pltpu.VMEM((GB, 128), jnp.int32),
        pltpu.VMEM((GB, 128, 16), jnp.float32),
        pltpu.VMEM((LANES,), jnp.int32),
        pltpu.SemaphoreType.DMA,
    ],
)(_sc_bin_body)


def _tc_comp_body(counts_ref, bg_ref, gp_ref, out_ref, accr, accg, accb, trans_ref):
    i = pl.program_id(0)
    strip = i // NSEG
    seg = lax.rem(i, NSEG)

    @pl.when(seg == 0)
    def _init():
        accr[:, :] = jnp.zeros((SH, W), jnp.float32)
        accg[:, :] = jnp.zeros((SH, W), jnp.float32)
        accb[:, :] = jnp.zeros((SH, W), jnp.float32)
        trans_ref[:, :] = jnp.ones((SH, W), jnp.float32)

    count = counts_ref[i, 0]
    xs = jax.lax.broadcasted_iota(jnp.int32, (1, 1, W), 2).astype(jnp.float32) + PIX_OFF
    ys = (jax.lax.broadcasted_iota(jnp.int32, (1, SH, 1), 1) + strip * SH
          ).astype(jnp.float32) + PIX_OFF

    def chunk(j, _):
        p = gp_ref[0, pl.ds(j * KC, KC), :]
        mx = p[:, 0:1][:, :, None]
        my = p[:, 1:2][:, :, None]
        a = p[:, 2:3][:, :, None]
        b = p[:, 3:4][:, :, None]
        c = p[:, 4:5][:, :, None]
        op = p[:, 5:6][:, :, None]
        tau = p[:, 9:10][:, :, None]

        rowmask = (j * KC + jax.lax.broadcasted_iota(jnp.int32, (KC, 1, 1), 0)) < count

        dx = xs - mx
        dy = ys - my
        q = a * (dx * dx) + 2.0 * b * (dx * dy) + c * (dy * dy)
        alpha = jnp.where((q <= tau) & rowmask, op * jnp.exp(-0.5 * q), 0.0)
        alpha = jnp.minimum(alpha, MAX_ALPHA)

        t = trans_ref[:, :]
        ws = []
        for g in range(KC):
            ag = alpha[g]
            ws.append(t * ag)
            t = t * (1.0 - ag)
        wstack = jnp.stack(ws, axis=0)

        cr = p[:, 6:7][:, :, None]
        cg = p[:, 7:8][:, :, None]
        cb = p[:, 8:9][:, :, None]
        accr[:, :] += jnp.sum(wstack * cr, axis=0)
        accg[:, :] += jnp.sum(wstack * cg, axis=0)
        accb[:, :] += jnp.sum(wstack * cb, axis=0)
        trans_ref[:, :] = t
        return 0

    lax.fori_loop(0, lax.div(count + (KC - 1), KC), chunk, 0)

    @pl.when(seg == NSEG - 1)
    def _fin():
        tt = trans_ref[:, :]
        out_ref[0, :, :] = accr[:, :] + tt * bg_ref[0]
        out_ref[1, :, :] = accg[:, :] + tt * bg_ref[1]
        out_ref[2, :, :] = accb[:, :] + tt * bg_ref[2]


def kernel(means2d, conics, colors, opacities, depths, background, image_height, image_width):
    order = jnp.argsort(lax.stop_gradient(depths))
    params = jnp.zeros((G, 16), jnp.float32)
    params = params.at[:, 0:2].set(means2d)
    params = params.at[:, 2:5].set(conics)
    params = params.at[:, 5].set(opacities)
    params = params.at[:, 6:9].set(colors)
    params = jnp.take(params, order, axis=0)

    prep = pl.pallas_call(
        _tc_prep_body,
        in_specs=[pl.BlockSpec((16, G), lambda: (0, 0))],
        out_specs=pl.BlockSpec((24, G), lambda: (0, 0)),
        out_shape=jax.ShapeDtypeStruct((24, G), jnp.float32),
    )(params.T)

    gp, counts = _sc_bin(prep, prep[0:16].T)
    gp = gp.reshape(NWORK, CAP, 16)

    out = pl.pallas_call(
        _tc_comp_body,
        grid=(NWORK,),
        in_specs=[
            pl.BlockSpec(memory_space=pltpu.SMEM),
            pl.BlockSpec(memory_space=pltpu.SMEM),
            pl.BlockSpec((1, CAP, 16), lambda i: (i, 0, 0)),
        ],
        out_specs=pl.BlockSpec((3, SH, W), lambda i: (0, i // NSEG, 0)),
        out_shape=jax.ShapeDtypeStruct((3, H, W), jnp.float32),
        scratch_shapes=[
            pltpu.VMEM((SH, W), jnp.float32),
            pltpu.VMEM((SH, W), jnp.float32),
            pltpu.VMEM((SH, W), jnp.float32),
            pltpu.VMEM((SH, W), jnp.float32),
        ],
    )(counts, background.astype(jnp.float32), gp)
    return jnp.transpose(out, (1, 2, 0)).astype(means2d.dtype)
